# baseline (speedup 1.0000x reference)
import jax
import jax.numpy as jnp
from jax import lax
from jax.experimental import pallas as pl
from jax.experimental.pallas import tpu as pltpu
from jax.experimental.pallas import tpu_sc as plsc

_NC = 2
_NS = 16
_L = 16
_NW = _NC * _NS
_B = 16384
_F = 32
_BPW = _B // _NW
_CHUNK = 128
_NCHUNK = _BPW // _CHUNK


def _body(user_hbm, item_hbm, uemb_hbm, iemb_hbm, out_hbm,
          uidx_v, iidx_v, urows_v, irows_v, out_v, sem):
    wid = lax.axis_index("s") * _NC + lax.axis_index("c")
    base = wid * _BPW

    pltpu.sync_copy(user_hbm.at[pl.ds(wid * _NCHUNK, _NCHUNK)], uidx_v)
    pltpu.sync_copy(item_hbm.at[pl.ds(wid * _NCHUNK, _NCHUNK)], iidx_v)

    copies = []
    for c in range(_NCHUNK):
        copies.append(pltpu.async_copy(
            uemb_hbm.at[uidx_v.at[c]], urows_v.at[pl.ds(c * _CHUNK, _CHUNK)], sem))
        copies.append(pltpu.async_copy(
            iemb_hbm.at[iidx_v.at[c]], irows_v.at[pl.ds(c * _CHUNK, _CHUNK)], sem))
    for cp in copies:
        cp.wait()

    lanes = lax.iota(jnp.int32, _L)

    def group(g, carry):
        row = g * _L + lanes
        acc = jnp.zeros((_L,), jnp.float32)
        for f in range(_F):
            col = jnp.full((_L,), f, jnp.int32)
            u = plsc.load_gather(urows_v, [row, col])
            v = plsc.load_gather(irows_v, [row, col])
            acc = acc + u * v
        out_v[pl.ds(g * _L, _L)] = 1.0 / (1.0 + jnp.exp(-acc))
        return carry

    lax.fori_loop(0, _BPW // _L, group, 0)

    pltpu.sync_copy(out_v, out_hbm.at[pl.ds(base, _BPW)])


def kernel(user, item, user_emb, item_emb):
    user2 = user.astype(jnp.int32).reshape(_B // _CHUNK, _CHUNK)
    item2 = item.astype(jnp.int32).reshape(_B // _CHUNK, _CHUNK)
    run = pl.kernel(
        _body,
        out_type=jax.ShapeDtypeStruct((_B,), jnp.float32),
        mesh=plsc.VectorSubcoreMesh(
            core_axis_name="c", subcore_axis_name="s",
            num_cores=_NC, num_subcores=_NS),
        scratch_types=[
            pltpu.VMEM((_NCHUNK, _CHUNK), jnp.int32),
            pltpu.VMEM((_NCHUNK, _CHUNK), jnp.int32),
            pltpu.VMEM((_BPW, _F), jnp.float32),
            pltpu.VMEM((_BPW, _F), jnp.float32),
            pltpu.VMEM((_BPW,), jnp.float32),
            pltpu.SemaphoreType.DMA,
        ],
        compiler_params=pltpu.CompilerParams(
            needs_layout_passes=False, use_tc_tiling_on_sc=False),
    )
    return run(user2, item2, user_emb, item_emb)
